# baseline (speedup 1.0000x reference)
import functools

import jax
import jax.numpy as jnp
from jax import lax
from jax.experimental import pallas as pl
from jax.experimental.pallas import tpu as pltpu
from jax.experimental.pallas import tpu_sc as plsc

NC = 2
NS = 16
NW = NC * NS

R = 4
C = 2048
B = R * C
D = 1024
CHUNK = 16
NBUF = 6
B_PER_W = B // NW
CHUNKS = [CHUNK] * (B_PER_W // CHUNK - 1) + [CHUNK // 2, CHUNK // 2]
OFFS = [sum(CHUNKS[:i]) for i in range(len(CHUNKS))]
NCH = len(CHUNKS)
W_PER_ROW = C // B_PER_W

_mesh = plsc.VectorSubcoreMesh(core_axis_name="c", subcore_axis_name="s")


@functools.partial(
    pl.kernel,
    out_type=jax.ShapeDtypeStruct((B, D), jnp.float32),
    mesh=_mesh,
    scratch_types=[
        pltpu.VMEM((B_PER_W,), jnp.int32),
        [pltpu.VMEM((CHUNK, D), jnp.float32) for _ in range(NBUF)],
        [pltpu.SemaphoreType.DMA for _ in range(NBUF)],
        [pltpu.SemaphoreType.DMA for _ in range(NBUF)],
    ],
)
def _gather_kernel(idx_hbm, table_hbm, out_hbm, idx_v, bufs, gsems, wsems):
    wid = lax.axis_index("s") * NC + lax.axis_index("c")
    base = wid * B_PER_W
    row = wid // W_PER_ROW
    col = (wid % W_PER_ROW) * B_PER_W
    pltpu.sync_copy(idx_hbm.at[row, pl.ds(col, B_PER_W)], idx_v)

    gd = [None] * NCH
    wd = [None] * NBUF
    LAG = NBUF - 3

    def write(k):
        b = k % NBUF
        ch = CHUNKS[k]
        gd[k].wait()
        wd[b] = pltpu.async_copy(
            bufs[b].at[pl.ds(0, ch)],
            out_hbm.at[pl.ds(base + OFFS[k], ch)],
            wsems[b],
        )

    for j in range(NCH):
        b = j % NBUF
        if wd[b] is not None:
            wd[b].wait()
        ch = CHUNKS[j]
        gd[j] = pltpu.async_copy(
            table_hbm.at[idx_v.at[pl.ds(OFFS[j], ch)]],
            bufs[b].at[pl.ds(0, ch)],
            gsems[b],
        )
        if j >= LAG:
            write(j - LAG)
    for k in range(NCH - LAG, NCH):
        write(k)
    for b in range(NBUF):
        if wd[b] is not None:
            wd[b].wait()


def kernel(x, table):
    out = _gather_kernel(x.astype(jnp.int32), table)
    return out.reshape(x.shape + (D,))


# speedup vs baseline: 1.0135x; 1.0135x over previous
import functools

import jax
import jax.numpy as jnp
from jax import lax
from jax.experimental import pallas as pl
from jax.experimental.pallas import tpu as pltpu
from jax.experimental.pallas import tpu_sc as plsc

NC = 2
NS = 16
NW = NC * NS

R = 4
C = 2048
B = R * C
D = 1024
CHUNK = 16
NBUF = 6
B_PER_W = B // NW
CHUNKS = [CHUNK] * (B_PER_W // CHUNK - 1) + [CHUNK // 2, CHUNK // 2]
OFFS = [sum(CHUNKS[:i]) for i in range(len(CHUNKS))]
NCH = len(CHUNKS)
W_PER_ROW = C // B_PER_W

_mesh = plsc.VectorSubcoreMesh(core_axis_name="c", subcore_axis_name="s")


@functools.partial(
    pl.kernel,
    out_type=jax.ShapeDtypeStruct((B, D), jnp.float32),
    mesh=_mesh,
    scratch_types=[
        pltpu.VMEM((B_PER_W,), jnp.int32),
        [pltpu.VMEM((CHUNK, D), jnp.float32) for _ in range(NBUF)],
        [pltpu.SemaphoreType.DMA for _ in range(NBUF)],
        [pltpu.SemaphoreType.DMA for _ in range(NBUF)],
    ],
)
def _gather_kernel(idx_hbm, table_hbm, out_hbm, idx_v, bufs, gsems, wsems):
    wid = lax.axis_index("s") * NC + lax.axis_index("c")
    base = wid * B_PER_W
    row = wid // W_PER_ROW
    col = (wid % W_PER_ROW) * B_PER_W
    pltpu.sync_copy(idx_hbm.at[row, pl.ds(col, B_PER_W)], idx_v)

    gd = [None] * NCH
    wd = [None] * NBUF
    LAG = NBUF - 2

    def write(k):
        b = k % NBUF
        ch = CHUNKS[k]
        gd[k].wait()
        wd[b] = pltpu.async_copy(
            bufs[b].at[pl.ds(0, ch)],
            out_hbm.at[pl.ds(base + OFFS[k], ch)],
            wsems[b],
        )

    for j in range(NCH):
        b = j % NBUF
        if wd[b] is not None:
            wd[b].wait()
        ch = CHUNKS[j]
        gd[j] = pltpu.async_copy(
            table_hbm.at[idx_v.at[pl.ds(OFFS[j], ch)]],
            bufs[b].at[pl.ds(0, ch)],
            gsems[b],
        )
        if j >= LAG:
            write(j - LAG)
    for k in range(NCH - LAG, NCH):
        write(k)
    for b in range(NBUF):
        if wd[b] is not None:
            wd[b].wait()


def kernel(x, table):
    out = _gather_kernel(x.astype(jnp.int32), table)
    return out.reshape(x.shape + (D,))
